# baseline (speedup 1.0000x reference)
import functools

import jax
import jax.numpy as jnp
from jax import lax
from jax.experimental import pallas as pl
from jax.experimental.pallas import tpu as pltpu
from jax.experimental.pallas import tpu_sc as plsc

BATCH = 16384
HIST = 200
EMBED = 32
VOCAB = 31
TOTAL = BATCH * HIST
LANES = 16
NW = 32

SC_FRAC_NUM = 1
SC_FRAC_DEN = 2
SC_TOTAL = (TOTAL * SC_FRAC_NUM // SC_FRAC_DEN // (NW * 2048)) * (NW * 2048)
TC_TOTAL = TOTAL - SC_TOTAL

CHUNK = 2048
TC_BLK = 16384


def _sc_embed(idx_flat, table_flat, n_total):
    per_tile = n_total // NW
    nchunk = per_tile // CHUNK
    mesh = plsc.VectorSubcoreMesh(core_axis_name="c", subcore_axis_name="s")

    @functools.partial(
        pl.kernel,
        mesh=mesh,
        out_type=jax.ShapeDtypeStruct((n_total * EMBED,), jnp.float32),
        scratch_types=[
            pltpu.VMEM((VOCAB * EMBED,), jnp.float32),
            pltpu.VMEM((CHUNK,), jnp.int32),
            pltpu.VMEM((CHUNK * EMBED,), jnp.float32),
        ],
        compiler_params=pltpu.CompilerParams(needs_layout_passes=False),
    )
    def k(idx_hbm, table_hbm, out_hbm, table_v, idx_v, rows_v):
        wid = lax.axis_index("s") * 2 + lax.axis_index("c")
        in_base = wid * per_tile
        out_base = in_base * EMBED
        pltpu.sync_copy(table_hbm, table_v)
        lane = lax.iota(jnp.int32, LANES)
        lane_off = lane * EMBED

        def chunk_body(i, _):
            pltpu.sync_copy(idx_hbm.at[pl.ds(in_base + i * CHUNK, CHUNK)],
                            idx_v)

            @plsc.parallel_loop(0, CHUNK // LANES, unroll=2)
            def group_body(g):
                iv = idx_v[pl.ds(g * LANES, LANES)]
                rb = iv * EMBED
                diff = g * (LANES * EMBED) + lane_off - rb
                for t in range(EMBED):
                    ga = rb + (lane ^ t)
                    vals = plsc.load_gather(table_v, [ga])
                    plsc.store_scatter(rows_v, [ga + diff], vals)

            pltpu.sync_copy(
                rows_v,
                out_hbm.at[pl.ds(out_base + i * CHUNK * EMBED,
                                 CHUNK * EMBED)])
            return ()

        lax.fori_loop(0, nchunk, chunk_body, ())

    return k(idx_flat, table_flat)


def _tc_body(idx_ref, tab_ref, out_ref):
    idx = idx_ref[0, 0, :]
    vocab_iota = lax.broadcasted_iota(jnp.int32, (TC_BLK, EMBED), 1)
    oh = (idx[:, None] == vocab_iota).astype(jnp.float32)
    out_ref[...] = jnp.dot(oh, tab_ref[...],
                           preferred_element_type=jnp.float32)


def _tc_embed(idx3d, table_pad, n_total):
    nblk = n_total // TC_BLK
    return pl.pallas_call(
        _tc_body,
        grid=(nblk,),
        in_specs=[
            pl.BlockSpec((1, 1, TC_BLK), lambda i: (i, 0, 0)),
            pl.BlockSpec((EMBED, EMBED), lambda i: (0, 0)),
        ],
        out_specs=pl.BlockSpec((TC_BLK, EMBED), lambda i: (i, 0)),
        out_shape=jax.ShapeDtypeStruct((n_total, EMBED), jnp.float32),
    )(idx3d, table_pad)


def kernel(monosaccharides, table):
    idx_flat = monosaccharides.reshape(TOTAL).astype(jnp.int32)
    sc_out = _sc_embed(idx_flat[:SC_TOTAL], table.reshape(VOCAB * EMBED),
                       SC_TOTAL)
    idx3d = idx_flat[SC_TOTAL:].reshape(TC_TOTAL // TC_BLK, 1, TC_BLK)
    table_pad = jnp.pad(table, ((0, EMBED - VOCAB), (0, 0)))
    tc_out = _tc_embed(idx3d, table_pad, TC_TOTAL)
    out = jnp.concatenate([sc_out.reshape(SC_TOTAL, EMBED), tc_out], axis=0)
    return out.reshape(BATCH, HIST, EMBED)


# speedup vs baseline: 4.9992x; 4.9992x over previous
import functools

import jax
import jax.numpy as jnp
from jax import lax
from jax.experimental import pallas as pl
from jax.experimental.pallas import tpu as pltpu
from jax.experimental.pallas import tpu_sc as plsc

BATCH = 16384
HIST = 200
EMBED = 32
VOCAB = 31
TOTAL = BATCH * HIST
LANES = 16
NW = 32
PER_TILE = TOTAL // NW
CHUNK = 1024
CN = CHUNK * EMBED
NCHUNK = PER_TILE // CHUNK


def _sc_embed(idx_flat, table_flat):
    mesh = plsc.VectorSubcoreMesh(core_axis_name="c", subcore_axis_name="s")

    @functools.partial(
        pl.kernel,
        mesh=mesh,
        out_type=jax.ShapeDtypeStruct((TOTAL * EMBED,), jnp.float32),
        scratch_types=[
            pltpu.VMEM((VOCAB * EMBED,), jnp.float32),
            pltpu.VMEM((CHUNK,), jnp.int32),
            pltpu.VMEM((2 * CN,), jnp.float32),
            pltpu.SemaphoreType.DMA((2,)),
        ],
        compiler_params=pltpu.CompilerParams(needs_layout_passes=False),
    )
    def k(idx_hbm, table_hbm, out_hbm, table_v, idx_v, rows_v, sem):
        wid = lax.axis_index("s") * 2 + lax.axis_index("c")
        in_base = wid * PER_TILE
        out_base = in_base * EMBED
        pltpu.sync_copy(table_hbm, table_v)
        lane = lax.iota(jnp.int32, LANES)
        lane_off = lane * EMBED

        def chunk_body(i, _):
            buf = lax.rem(i, 2)
            pltpu.sync_copy(idx_hbm.at[pl.ds(in_base + i * CHUNK, CHUNK)],
                            idx_v)

            @pl.when(i >= 2)
            def _():
                pltpu.make_async_copy(
                    rows_v.at[pl.ds(buf * CN, CN)],
                    out_hbm.at[pl.ds(0, CN)], sem.at[buf]).wait()

            @plsc.parallel_loop(0, CHUNK // LANES, unroll=2)
            def group_body(g):
                iv = idx_v[pl.ds(g * LANES, LANES)]
                rb = iv * EMBED
                diff = buf * CN + g * (LANES * EMBED) + lane_off - rb
                for t in range(EMBED):
                    ga = rb + (lane ^ t)
                    vals = plsc.load_gather(table_v, [ga])
                    plsc.store_scatter(rows_v, [ga + diff], vals)

            pltpu.async_copy(
                rows_v.at[pl.ds(buf * CN, CN)],
                out_hbm.at[pl.ds(out_base + i * CN, CN)],
                sem.at[buf])
            return ()

        lax.fori_loop(0, NCHUNK, chunk_body, ())
        for b in range(2):
            pltpu.make_async_copy(
                rows_v.at[pl.ds(b * CN, CN)],
                out_hbm.at[pl.ds(0, CN)], sem.at[b]).wait()

    return k(idx_flat, table_flat)


def kernel(monosaccharides, table):
    idx_flat = monosaccharides.reshape(TOTAL).astype(jnp.int32)
    out = _sc_embed(idx_flat, table.reshape(VOCAB * EMBED))
    return out.reshape(BATCH, HIST, EMBED)
